# Initial kernel scaffold; baseline (speedup 1.0000x reference)
import jax
import jax.numpy as jnp
from jax.experimental import pallas as pl


def kernel(x, pos_x, pos_y, batch_x, batch_y, k):
    raise NotImplementedError("write your pallas kernel here")


# speedup vs baseline: 3.0350x; 3.0350x over previous
import functools

import jax
import jax.numpy as jnp
from jax.experimental import pallas as pl
from jax.experimental.pallas import tpu as pltpu

_MB = 128
_TN = 2048
_NEG = float("-inf")
_INF = float("inf")


def _knn_interp_kernel(k_ref, pos_y_ref, by_ref, pos_xt_ref, bx_ref, x_ref,
                       out_ref, d2_ref, *, n, d):
    nt = n // _TN
    kdyn = k_ref[0]
    yx = pos_y_ref[:, 0:1]
    yy = pos_y_ref[:, 1:2]
    yz = pos_y_ref[:, 2:3]
    by = by_ref[:, 0:1]
    ynorm = yx * yx + yy * yy + yz * yz
    yb = pos_y_ref[...].astype(jnp.bfloat16)

    def _dist(i, carry):
        off = i * _TN
        xs = pos_xt_ref[:, pl.ds(off, _TN)]
        x0 = xs[0:1, :]
        x1 = xs[1:2, :]
        x2 = xs[2:3, :]
        xnorm = x0 * x0 + x1 * x1 + x2 * x2
        mm = jnp.dot(yb, xs.astype(jnp.bfloat16),
                     preferred_element_type=jnp.float32)
        d2 = (ynorm + xnorm) - 2.0 * mm
        valid = by == bx_ref[:, pl.ds(off, _TN)]
        d2_ref[:, pl.ds(off, _TN)] = jnp.where(valid, d2, _INF)
        return carry

    jax.lax.fori_loop(0, nt, _dist, 0, unroll=2)

    t = jnp.full((_MB, 1), _NEG, dtype=jnp.float32)
    t_sel = jnp.full((_MB, 1), _NEG, dtype=jnp.float32)
    for r in range(8):
        def _round(i, m):
            v = d2_ref[:, pl.ds(i * _TN, _TN)]
            vm = jnp.where(v <= t, _INF, v)
            return jnp.minimum(m, jnp.min(vm, axis=1, keepdims=True))

        t = jax.lax.fori_loop(0, nt, _round,
                              jnp.full((_MB, 1), _INF, dtype=jnp.float32),
                              unroll=2)
        t_sel = jnp.where(r < kdyn, t, t_sel)
    t = t_sel

    def _combine(i, carry):
        num, den = carry
        off = i * _TN
        v = d2_ref[:, pl.ds(off, _TN)]
        xs = pos_xt_ref[:, pl.ds(off, _TN)]
        d2e = ((yx - xs[0:1, :]) ** 2
               + (yy - xs[1:2, :]) ** 2
               + (yz - xs[2:3, :]) ** 2)
        w = jnp.where(v <= t, 1.0 / jnp.maximum(d2e, 1e-16), 0.0)
        den = den + jnp.sum(w, axis=1, keepdims=True)
        num = num + jnp.dot(w, x_ref[pl.ds(off, _TN), :],
                            preferred_element_type=jnp.float32,
                            precision=jax.lax.Precision.HIGHEST)
        return num, den

    num0 = jnp.zeros((_MB, d), dtype=jnp.float32)
    den0 = jnp.zeros((_MB, 1), dtype=jnp.float32)
    num, den = jax.lax.fori_loop(0, nt, _combine, (num0, den0), unroll=2)
    out_ref[...] = num / den


def kernel(x, pos_x, pos_y, batch_x, batch_y, k):
    n, d = x.shape
    m = pos_y.shape[0]
    karr = jnp.asarray(k, dtype=jnp.int32).reshape(1)
    pos_xt = pos_x.T
    bx = batch_x.astype(jnp.int32).reshape(1, n)
    by = batch_y.astype(jnp.int32).reshape(m, 1)

    grid = (m // _MB,)
    out = pl.pallas_call(
        functools.partial(_knn_interp_kernel, n=n, d=d),
        grid=grid,
        in_specs=[
            pl.BlockSpec(memory_space=pltpu.SMEM),
            pl.BlockSpec((_MB, 3), lambda i: (i, 0)),
            pl.BlockSpec((_MB, 1), lambda i: (i, 0)),
            pl.BlockSpec((3, n), lambda i: (0, 0)),
            pl.BlockSpec((1, n), lambda i: (0, 0)),
            pl.BlockSpec((n, d), lambda i: (0, 0)),
        ],
        out_specs=pl.BlockSpec((_MB, d), lambda i: (i, 0)),
        out_shape=jax.ShapeDtypeStruct((m, d), jnp.float32),
        scratch_shapes=[pltpu.VMEM((_MB, n), jnp.float32)],
    )(karr, pos_y, by, pos_xt, bx, x)
    return out


# speedup vs baseline: 3.2815x; 1.0812x over previous
import functools

import jax
import jax.numpy as jnp
from jax.experimental import pallas as pl
from jax.experimental.pallas import tpu as pltpu

_MB = 128
_TN = 2048
_NEG = float("-inf")
_INF = float("inf")


def _knn_interp_kernel(k_ref, pos_y_ref, by_ref, pos_xt_ref, bx_ref, x_ref,
                       out_ref, d2_ref, *, n, d):
    nt = n // _TN
    kdyn = k_ref[0]
    yx = pos_y_ref[:, 0:1]
    yy = pos_y_ref[:, 1:2]
    yz = pos_y_ref[:, 2:3]
    by = by_ref[:, 0:1]
    ynorm = yx * yx + yy * yy + yz * yz
    yb = pos_y_ref[...].astype(jnp.bfloat16)

    def _dist(i, m):
        off = i * _TN
        xs = pos_xt_ref[:, pl.ds(off, _TN)]
        x0 = xs[0:1, :]
        x1 = xs[1:2, :]
        x2 = xs[2:3, :]
        xnorm = x0 * x0 + x1 * x1 + x2 * x2
        mm = jnp.dot(yb, xs.astype(jnp.bfloat16),
                     preferred_element_type=jnp.float32)
        d2 = (ynorm + xnorm) - 2.0 * mm
        valid = by == bx_ref[:, pl.ds(off, _TN)]
        d2m = jnp.where(valid, d2, _INF)
        d2_ref[:, pl.ds(off, _TN)] = d2m
        return jnp.minimum(m, jnp.min(d2m, axis=1, keepdims=True))

    t = jax.lax.fori_loop(0, nt, _dist,
                          jnp.full((_MB, 1), _INF, dtype=jnp.float32),
                          unroll=2)

    t_sel = jnp.where(0 < kdyn, t, _NEG)
    for r in range(1, 8):
        def _round(i, m):
            v = d2_ref[:, pl.ds(i * _TN, _TN)]
            vm = jnp.where(v <= t, _INF, v)
            return jnp.minimum(m, jnp.min(vm, axis=1, keepdims=True))

        t = jax.lax.fori_loop(0, nt, _round,
                              jnp.full((_MB, 1), _INF, dtype=jnp.float32),
                              unroll=4)
        t_sel = jnp.where(r < kdyn, t, t_sel)
    t = t_sel

    def _combine(i, carry):
        num, den = carry
        off = i * _TN
        v = d2_ref[:, pl.ds(off, _TN)]
        xs = pos_xt_ref[:, pl.ds(off, _TN)]
        d2e = ((yx - xs[0:1, :]) ** 2
               + (yy - xs[1:2, :]) ** 2
               + (yz - xs[2:3, :]) ** 2)
        w = jnp.where(v <= t, 1.0 / jnp.maximum(d2e, 1e-16), 0.0)
        den = den + jnp.sum(w, axis=1, keepdims=True)
        num = num + jnp.dot(w, x_ref[pl.ds(off, _TN), :],
                            preferred_element_type=jnp.float32,
                            precision=jax.lax.Precision.HIGHEST)
        return num, den

    num0 = jnp.zeros((_MB, d), dtype=jnp.float32)
    den0 = jnp.zeros((_MB, 1), dtype=jnp.float32)
    num, den = jax.lax.fori_loop(0, nt, _combine, (num0, den0), unroll=2)
    out_ref[...] = num / den


def kernel(x, pos_x, pos_y, batch_x, batch_y, k):
    n, d = x.shape
    m = pos_y.shape[0]
    karr = jnp.asarray(k, dtype=jnp.int32).reshape(1)
    pos_xt = pos_x.T
    bx = batch_x.astype(jnp.int32).reshape(1, n)
    by = batch_y.astype(jnp.int32).reshape(m, 1)

    grid = (m // _MB,)
    out = pl.pallas_call(
        functools.partial(_knn_interp_kernel, n=n, d=d),
        grid=grid,
        in_specs=[
            pl.BlockSpec(memory_space=pltpu.SMEM),
            pl.BlockSpec((_MB, 3), lambda i: (i, 0)),
            pl.BlockSpec((_MB, 1), lambda i: (i, 0)),
            pl.BlockSpec((3, n), lambda i: (0, 0)),
            pl.BlockSpec((1, n), lambda i: (0, 0)),
            pl.BlockSpec((n, d), lambda i: (0, 0)),
        ],
        out_specs=pl.BlockSpec((_MB, d), lambda i: (i, 0)),
        out_shape=jax.ShapeDtypeStruct((m, d), jnp.float32),
        scratch_shapes=[pltpu.VMEM((_MB, n), jnp.float32)],
    )(karr, pos_y, by, pos_xt, bx, x)
    return out


# speedup vs baseline: 3.9007x; 1.1887x over previous
import functools

import jax
import jax.numpy as jnp
from jax.experimental import pallas as pl
from jax.experimental.pallas import tpu as pltpu

_MB = 128
_TN = 2048
_NEG = float("-inf")
_INF = float("inf")


def _lane_fold(v):
    parts = [v[:, j * 128:(j + 1) * 128] for j in range(v.shape[1] // 128)]
    acc = parts[0]
    for p in parts[1:]:
        acc = jnp.minimum(acc, p)
    return acc


def _knn_interp_kernel(k_ref, pos_y_ref, by_ref, pos_xt_ref, bx_ref,
                       xh_ref, xl_ref, out_ref, d2_ref, *, n, d):
    nt = n // _TN
    kdyn = k_ref[0]
    yx = pos_y_ref[:, 0:1]
    yy = pos_y_ref[:, 1:2]
    yz = pos_y_ref[:, 2:3]
    by = by_ref[:, 0:1]
    ynorm = yx * yx + yy * yy + yz * yz
    yb = pos_y_ref[...].astype(jnp.bfloat16)

    def _dist(i, m):
        off = i * _TN
        xs = pos_xt_ref[:, pl.ds(off, _TN)]
        x0 = xs[0:1, :]
        x1 = xs[1:2, :]
        x2 = xs[2:3, :]
        xnorm = x0 * x0 + x1 * x1 + x2 * x2
        mm = jnp.dot(yb, xs.astype(jnp.bfloat16),
                     preferred_element_type=jnp.float32)
        d2 = (ynorm + xnorm) - 2.0 * mm
        valid = by == bx_ref[:, pl.ds(off, _TN)]
        d2m = jnp.where(valid, d2, _INF)
        d2_ref[:, pl.ds(off, _TN)] = d2m
        return jnp.minimum(m, _lane_fold(d2m))

    acc = jax.lax.fori_loop(0, nt, _dist,
                            jnp.full((_MB, 128), _INF, dtype=jnp.float32),
                            unroll=2)
    t = jnp.min(acc, axis=1, keepdims=True)

    t_sel = jnp.where(0 < kdyn, t, _NEG)
    for r in range(1, 8):
        def _round(i, m):
            v = d2_ref[:, pl.ds(i * _TN, _TN)]
            vm = jnp.where(v <= t, _INF, v)
            return jnp.minimum(m, _lane_fold(vm))

        acc = jax.lax.fori_loop(0, nt, _round,
                                jnp.full((_MB, 128), _INF, dtype=jnp.float32),
                                unroll=4)
        t = jnp.min(acc, axis=1, keepdims=True)
        t_sel = jnp.where(r < kdyn, t, t_sel)
    t = t_sel

    def _combine(i, carry):
        num, den = carry
        off = i * _TN
        v = d2_ref[:, pl.ds(off, _TN)]
        xs = pos_xt_ref[:, pl.ds(off, _TN)]
        d2e = ((yx - xs[0:1, :]) ** 2
               + (yy - xs[1:2, :]) ** 2
               + (yz - xs[2:3, :]) ** 2)
        w = jnp.where(v <= t, 1.0 / jnp.maximum(d2e, 1e-16), 0.0)
        den = den + jnp.sum(w, axis=1, keepdims=True)
        w_hi = w.astype(jnp.bfloat16)
        w_lo = (w - w_hi.astype(jnp.float32)).astype(jnp.bfloat16)
        xh = xh_ref[pl.ds(off, _TN), :]
        xl = xl_ref[pl.ds(off, _TN), :]
        num = (num
               + jnp.dot(w_hi, xh, preferred_element_type=jnp.float32)
               + jnp.dot(w_hi, xl, preferred_element_type=jnp.float32)
               + jnp.dot(w_lo, xh, preferred_element_type=jnp.float32))
        return num, den

    num0 = jnp.zeros((_MB, d), dtype=jnp.float32)
    den0 = jnp.zeros((_MB, 1), dtype=jnp.float32)
    num, den = jax.lax.fori_loop(0, nt, _combine, (num0, den0), unroll=2)
    out_ref[...] = num / den


def kernel(x, pos_x, pos_y, batch_x, batch_y, k):
    n, d = x.shape
    m = pos_y.shape[0]
    karr = jnp.asarray(k, dtype=jnp.int32).reshape(1)
    pos_xt = pos_x.T
    bx = batch_x.astype(jnp.int32).reshape(1, n)
    by = batch_y.astype(jnp.int32).reshape(m, 1)
    x_hi = x.astype(jnp.bfloat16)
    x_lo = (x - x_hi.astype(jnp.float32)).astype(jnp.bfloat16)

    grid = (m // _MB,)
    out = pl.pallas_call(
        functools.partial(_knn_interp_kernel, n=n, d=d),
        grid=grid,
        in_specs=[
            pl.BlockSpec(memory_space=pltpu.SMEM),
            pl.BlockSpec((_MB, 3), lambda i: (i, 0)),
            pl.BlockSpec((_MB, 1), lambda i: (i, 0)),
            pl.BlockSpec((3, n), lambda i: (0, 0)),
            pl.BlockSpec((1, n), lambda i: (0, 0)),
            pl.BlockSpec((n, d), lambda i: (0, 0)),
            pl.BlockSpec((n, d), lambda i: (0, 0)),
        ],
        out_specs=pl.BlockSpec((_MB, d), lambda i: (i, 0)),
        out_shape=jax.ShapeDtypeStruct((m, d), jnp.float32),
        scratch_shapes=[pltpu.VMEM((_MB, n), jnp.float32)],
    )(karr, pos_y, by, pos_xt, bx, x_hi, x_lo)
    return out
